# Initial kernel scaffold; baseline (speedup 1.0000x reference)
import jax
import jax.numpy as jnp
from jax.experimental import pallas as pl


def kernel(nodes, edges, receivers, senders, global_latent, node_graph_idx, edge_graph_idx, W, b):
    raise NotImplementedError("write your pallas kernel here")


# speedup vs baseline: 6.6625x; 6.6625x over previous
import functools

import jax
import jax.numpy as jnp
from jax import lax
from jax.experimental import pallas as pl
from jax.experimental.pallas import tpu as pltpu
from jax.experimental.pallas import tpu_sc as plsc

B = 4
N = 25000
D = 128
G = 256
TILE = 200
NT = N // TILE
CA, CB = 120, 80
NSUB = 8
JMAX = (NT + NSUB - 1) // NSUB


def _sc_segment_sum(nodes, idx, zeros):
    mesh = plsc.VectorSubcoreMesh(core_axis_name="c", subcore_axis_name="s")

    @functools.partial(
        pl.kernel,
        out_type=jax.ShapeDtypeStruct((B, G, D), jnp.float32),
        mesh=mesh,
        scratch_types=[
            pltpu.VMEM((2, TILE, D), jnp.float32),
            pltpu.VMEM((2, CA), jnp.int32),
            pltpu.VMEM((2, CB), jnp.int32),
            pltpu.VMEM_SHARED((G, D), jnp.float32),
            pltpu.VMEM_SHARED((G, D), jnp.float32),
            pltpu.SemaphoreType.DMA,
            pltpu.SemaphoreType.DMA,
            pltpu.SemaphoreType.DMA,
            pltpu.SemaphoreType.DMA,
        ],
    )
    def seg_kernel(nodes_h, idx_h, zeros_h, out_h,
                   nbuf, ia, ib, acc0, acc1, semn0, semn1, semi0, semi1):
        c = lax.axis_index("c")
        s = lax.axis_index("s")
        batch = 2 * c + s // NSUB
        wb = s % NSUB
        lb = s // NSUB
        semn = (semn0, semn1)
        semi = (semi0, semi1)

        @pl.when(s == 0)
        def _():
            pltpu.sync_copy(zeros_h, acc0)

        @pl.when(s == NSUB)
        def _():
            pltpu.sync_copy(zeros_h, acc1)

        plsc.subcore_barrier()

        def copies(j, slot):
            base = (wb + NSUB * j) * TILE
            fbase = batch * N + base
            return (
                pltpu.make_async_copy(
                    nodes_h.at[batch, pl.ds(base, TILE)], nbuf.at[slot], semn[slot]),
                pltpu.make_async_copy(
                    idx_h.at[pl.ds(fbase, CA)], ia.at[slot], semi[slot]),
                pltpu.make_async_copy(
                    idx_h.at[pl.ds(fbase + CA, CB)], ib.at[slot], semi[slot]),
            )

        def issue(j, slot):
            @pl.when(wb + NSUB * j < NT)
            def _():
                for d in copies(j, slot):
                    d.start()

        issue(0, 0)
        issue(1, 1)

        def body(jo, carry):
            for slot in range(2):
                j = 2 * jo + slot

                @pl.when(wb + NSUB * j < NT)
                def _(j=j, slot=slot):
                    for d in copies(j, slot):
                        d.wait()

                    @pl.when(lb == 0)
                    def _():
                        pltpu.sync_copy(nbuf.at[slot, pl.ds(0, CA)],
                                        acc0.at[ia.at[slot]], add=True)
                        pltpu.sync_copy(nbuf.at[slot, pl.ds(CA, CB)],
                                        acc0.at[ib.at[slot]], add=True)

                    @pl.when(lb == 1)
                    def _():
                        pltpu.sync_copy(nbuf.at[slot, pl.ds(0, CA)],
                                        acc1.at[ia.at[slot]], add=True)
                        pltpu.sync_copy(nbuf.at[slot, pl.ds(CA, CB)],
                                        acc1.at[ib.at[slot]], add=True)

                    issue(j + 2, slot)
            return carry

        lax.fori_loop(0, JMAX // 2, body, 0)
        plsc.subcore_barrier()

        @pl.when(s == 0)
        def _():
            pltpu.sync_copy(acc0, out_h.at[2 * c])

        @pl.when(s == NSUB)
        def _():
            pltpu.sync_copy(acc1, out_h.at[2 * c + 1])

    return seg_kernel(nodes, idx, zeros)


def _tc_head(seg, gl, W, b):
    def head_kernel(seg_ref, gl_ref, w_ref, b_ref, out_ref):
        w = w_ref[...]
        w1 = w[0:D, 0]
        w2 = w[D:2 * D, 0]
        bias = b_ref[0]
        for i in range(B):
            r = (jnp.sum(seg_ref[i] * w1[None, :], axis=-1)
                 + jnp.sum(gl_ref[i] * w2[None, :], axis=-1) + bias)
            out_ref[i] = r

    return pl.pallas_call(
        head_kernel,
        out_shape=jax.ShapeDtypeStruct((B, G), jnp.float32),
        in_specs=[
            pl.BlockSpec(memory_space=pltpu.MemorySpace.VMEM),
            pl.BlockSpec(memory_space=pltpu.MemorySpace.VMEM),
            pl.BlockSpec(memory_space=pltpu.MemorySpace.VMEM),
            pl.BlockSpec(memory_space=pltpu.MemorySpace.SMEM),
        ],
        out_specs=pl.BlockSpec(memory_space=pltpu.MemorySpace.VMEM),
    )(seg, gl, W, b)


def kernel(nodes, edges, receivers, senders, global_latent, node_graph_idx,
           edge_graph_idx, W, b):
    zeros = jnp.zeros((G, D), dtype=jnp.float32)
    seg = _sc_segment_sum(nodes, node_graph_idx.reshape(-1), zeros)
    out = _tc_head(seg, global_latent, W, b)
    return out.reshape(B, G, 1)
